# Initial kernel scaffold; baseline (speedup 1.0000x reference)
import jax
import jax.numpy as jnp
from jax.experimental import pallas as pl


def kernel(lprobs, scores, step):
    raise NotImplementedError("write your pallas kernel here")


# speedup vs baseline: 28.6445x; 28.6445x over previous
import functools

import jax
import jax.numpy as jnp
from jax import lax
from jax.experimental import pallas as pl
from jax.experimental.pallas import tpu as pltpu
from jax.experimental.pallas import tpu_sc as plsc

DIVERSITY_RATE = 0.5
NC = 2
NS = 16
L = 16

V = 100000
CHUNK = V // 2
BLK_VECS = 25
BLK = BLK_VECS * L
NBLK = CHUNK // BLK
K = 16


def _iota16():
    return lax.iota(jnp.int32, L)


_GATHER_DNUMS = lax.GatherDimensionNumbers(
    offset_dims=(), collapsed_slice_dims=(0,), start_index_map=(0,))


def _perm(x, idx):
    return lax.gather(x, idx[:, None], _GATHER_DNUMS, (1,),
                      mode=lax.GatherScatterMode.PROMISE_IN_BOUNDS)


def _lex_gt(av, ai, bv, bi):
    return (av > bv) | ((av == bv) & (ai < bi))


def _merge_top16(rv, ri, xv, xi):
    sv, si = plsc.sort_key_val(xv, xi, descending=False)
    g = _lex_gt(rv, ri, sv, si)
    hv = jnp.where(g, rv, sv)
    hi = jnp.where(g, ri, si)
    lanes = _iota16()
    for d in (8, 4, 2, 1):
        p = lanes ^ d
        pv = _perm(hv, p)
        pi = _perm(hi, p)
        gt = _lex_gt(hv, hi, pv, pi)
        want_max = (lanes & d) == 0
        keep = gt == want_max
        hv = jnp.where(keep, hv, pv)
        hi = jnp.where(keep, hi, pi)
    thr = _perm(hv, jnp.full((L,), 15, jnp.int32))
    return hv, hi, thr


def _tree_max(xs):
    while len(xs) > 1:
        xs = [jnp.maximum(xs[i], xs[i + 1]) for i in range(0, len(xs) - 1, 2)] \
            + ([xs[-1]] if len(xs) % 2 else [])
    return xs[0]


def _scan_chunk(dbuf, parity, bias, rv, ri, thr):
    off = parity * CHUNK

    def blk_body(blk, carry):
        rv, ri, thr = carry
        base = off + blk * BLK
        xs = [dbuf[pl.ds(base + i * L, L)] for i in range(BLK_VECS)]
        m = _tree_max(xs)
        pred = jnp.any(m + bias > thr)

        def slow(carry):
            def vec_body(i, carry):
                rv, ri, thr = carry
                xb = dbuf[pl.ds(base + i * L, L)] + bias
                def do_merge(c):
                    rv, ri, _ = c
                    idxv = jnp.broadcast_to(base + i * L, (L,)).astype(jnp.int32) \
                        + _iota16()
                    return _merge_top16(rv, ri, xb, idxv)
                return lax.cond(jnp.any(xb > thr), do_merge, lambda c: c,
                                (rv, ri, thr))
            return lax.fori_loop(0, BLK_VECS, vec_body, carry)

        return lax.cond(pred, slow, lambda c: c, (rv, ri, thr))

    return lax.fori_loop(0, NBLK, blk_body, (rv, ri, thr))


def _sc_kernel(lp_ref, bias_ref, os_ref, oi_ref, ob_ref,
               dbuf, biasv, valbuf, idxbuf, ov, oi, ob, sem0, sem1):
    wid = lax.axis_index("c") * NS + lax.axis_index("s")
    pltpu.sync_copy(bias_ref, biasv)
    row0 = wid * 8

    pltpu.async_copy(lp_ref.at[row0, pl.ds(0, CHUNK)],
                     dbuf.at[pl.ds(0, CHUNK)], sem0)

    neg_inf = jnp.full((L,), -jnp.inf, jnp.float32)
    zeros_i = jnp.zeros((L,), jnp.int32)

    def row_body(r, dummy):
        row = row0 + r
        bias = plsc.load_gather(biasv, [jnp.broadcast_to(row, (L,))
                                        .astype(jnp.int32)])
        pltpu.async_copy(lp_ref.at[row, pl.ds(CHUNK, CHUNK)],
                         dbuf.at[pl.ds(CHUNK, CHUNK)], sem1)
        pltpu.make_async_copy(lp_ref.at[row, pl.ds(0, CHUNK)],
                              dbuf.at[pl.ds(0, CHUNK)], sem0).wait()
        rv, ri, thr = _scan_chunk(dbuf, 0, bias, neg_inf, zeros_i, neg_inf)
        nrow = row0 + ((r + 1) & 7)
        pltpu.async_copy(lp_ref.at[nrow, pl.ds(0, CHUNK)],
                         dbuf.at[pl.ds(0, CHUNK)], sem0)
        pltpu.make_async_copy(lp_ref.at[row, pl.ds(CHUNK, CHUNK)],
                              dbuf.at[pl.ds(CHUNK, CHUNK)], sem1).wait()
        rv, ri, thr = _scan_chunk(dbuf, 1, bias, rv, ri, thr)
        valbuf[pl.ds(r * K, K)] = rv
        idxbuf[pl.ds(r * K, K)] = ri
        return dummy

    lax.fori_loop(0, 8, row_body, jnp.int32(0))
    pltpu.make_async_copy(lp_ref.at[row0, pl.ds(0, CHUNK)],
                          dbuf.at[pl.ds(0, CHUNK)], sem0).wait()

    pen = (_iota16().astype(jnp.float32) + 1.0) * DIVERSITY_RATE
    r2v, r2i = neg_inf, zeros_i
    for b in range(8):
        pv = valbuf[pl.ds(b * K, K)] - pen
        fi = jnp.broadcast_to(jnp.int32(b * K), (L,)) + _iota16()
        r2v, r2i, _ = _merge_top16(r2v, r2i, pv, fi)

    ov[...] = r2v
    oi[...] = plsc.load_gather(idxbuf, [r2i])
    ob[...] = jnp.right_shift(r2i, 4)
    pltpu.sync_copy(ov, os_ref.at[wid])
    pltpu.sync_copy(oi, oi_ref.at[wid])
    pltpu.sync_copy(ob, ob_ref.at[wid])


def kernel(lprobs, scores, step):
    bsz, beam, vocab = lprobs.shape
    assert (bsz, beam, vocab) == (32, 8, V)
    lp2 = lprobs.reshape(bsz * beam, vocab)
    bias = lax.dynamic_index_in_dim(scores, step - 1, axis=2,
                                    keepdims=False).reshape(bsz * beam)

    mesh = plsc.VectorSubcoreMesh(core_axis_name="c", subcore_axis_name="s",
                                  num_cores=NC, num_subcores=NS)
    f = pl.kernel(
        _sc_kernel,
        out_type=(
            jax.ShapeDtypeStruct((bsz, K), jnp.float32),
            jax.ShapeDtypeStruct((bsz, K), jnp.int32),
            jax.ShapeDtypeStruct((bsz, K), jnp.int32),
        ),
        mesh=mesh,
        compiler_params=pltpu.CompilerParams(use_tc_tiling_on_sc=False,
                                             needs_layout_passes=False),
        scratch_types=[
            pltpu.VMEM((V,), jnp.float32),
            pltpu.VMEM((bsz * beam,), jnp.float32),
            pltpu.VMEM((beam * K,), jnp.float32),
            pltpu.VMEM((beam * K,), jnp.int32),
            pltpu.VMEM((K,), jnp.float32),
            pltpu.VMEM((K,), jnp.int32),
            pltpu.VMEM((K,), jnp.int32),
            pltpu.SemaphoreType.DMA,
            pltpu.SemaphoreType.DMA,
        ],
    )
    return f(lp2, bias)


# speedup vs baseline: 47.1690x; 1.6467x over previous
import functools

import jax
import jax.numpy as jnp
from jax import lax
from jax.experimental import pallas as pl
from jax.experimental.pallas import tpu as pltpu
from jax.experimental.pallas import tpu_sc as plsc

DIVERSITY_RATE = 0.5
NC = 2
NS = 16
L = 16

V = 100000
CHUNK = V // 2
BLK_VECS = 25
BLK = BLK_VECS * L
NBLK = CHUNK // BLK
K = 16


def _iota16():
    return lax.iota(jnp.int32, L)


_GATHER_DNUMS = lax.GatherDimensionNumbers(
    offset_dims=(), collapsed_slice_dims=(0,), start_index_map=(0,))


def _perm(x, idx):
    return lax.gather(x, idx[:, None], _GATHER_DNUMS, (1,),
                      mode=lax.GatherScatterMode.PROMISE_IN_BOUNDS)


def _lex_gt(av, ai, bv, bi):
    return (av > bv) | ((av == bv) & (ai < bi))


def _merge_top16(rv, ri, xv, xi):
    sv, si = plsc.sort_key_val(xv, xi, descending=False)
    g = _lex_gt(rv, ri, sv, si)
    hv = jnp.where(g, rv, sv)
    hi = jnp.where(g, ri, si)
    lanes = _iota16()
    for d in (8, 4, 2, 1):
        p = lanes ^ d
        pv = _perm(hv, p)
        pi = _perm(hi, p)
        gt = _lex_gt(hv, hi, pv, pi)
        want_max = (lanes & d) == 0
        keep = gt == want_max
        hv = jnp.where(keep, hv, pv)
        hi = jnp.where(keep, hi, pi)
    thr = _perm(hv, jnp.full((L,), 15, jnp.int32))
    return hv, hi, thr


def _tree_max(xs):
    while len(xs) > 1:
        xs = [jnp.maximum(xs[i], xs[i + 1]) for i in range(0, len(xs) - 1, 2)] \
            + ([xs[-1]] if len(xs) % 2 else [])
    return xs[0]


def _count_ge(x, thr_v):
    cnt = plsc.all_reduce_population_count(x >= thr_v)
    return cnt[0]


def _scan_chunk(dbuf, bmaxbuf, parity, bias, carry):
    off = parity * CHUNK
    neg_inf_v = jnp.full((L,), -jnp.inf, jnp.float32)

    def a_body(blk, dummy):
        base = off + blk * BLK
        xs = [dbuf[pl.ds(base + i * L, L)] for i in range(BLK_VECS)]
        m = _tree_max(xs) + bias
        bmaxbuf[pl.ds(blk * L, L)] = plsc.cummax(m)
        return dummy

    lax.fori_loop(0, NBLK, a_body, jnp.int32(0), unroll=2)
    for pb in (125, 126, 127):
        bmaxbuf[pl.ds(pb * L, L)] = neg_inf_v

    rv, ri, thr_v = carry
    tv = neg_inf_v
    tb = jnp.zeros((L,), jnp.int32)
    for g in range(8):
        idxs = jnp.broadcast_to(jnp.int32(g * 256 + 15), (L,)) + _iota16() * L
        bm = plsc.load_gather(bmaxbuf, [idxs])
        tv, tb, _ = _merge_top16(tv, tb, bm,
                                 jnp.broadcast_to(jnp.int32(g * L), (L,))
                                 + _iota16())

    nblk = _count_ge(tv, thr_v)

    def rescan(j, carry):
        rv, ri, thr_v = carry
        blk = _perm(tb, jnp.broadcast_to(j, (L,)).astype(jnp.int32))[0]
        base = off + blk * BLK

        def v_body(i, carry):
            rv, ri, thr_v = carry
            pos = base + i * L
            xb = dbuf[pl.ds(pos, L)] + bias

            def do_merge(c2):
                rv, ri, _ = c2
                idxv = jnp.broadcast_to(pos, (L,)).astype(jnp.int32) + _iota16()
                return _merge_top16(rv, ri, xb, idxv)

            return lax.cond(_count_ge(xb, thr_v) > 0, do_merge,
                            lambda c2: c2, (rv, ri, thr_v))

        return lax.fori_loop(0, BLK_VECS, v_body, (rv, ri, thr_v), unroll=5)

    return lax.fori_loop(0, nblk, rescan, (rv, ri, thr_v))


NSTREAM = 10
SUB = CHUNK // NSTREAM


def _issue_chunk(lp_ref, dbuf, sem, row, src_chunk, dst_parity):
    for s in range(NSTREAM):
        pltpu.async_copy(
            lp_ref.at[row, pl.ds(src_chunk * CHUNK + s * SUB, SUB)],
            dbuf.at[pl.ds(dst_parity * CHUNK + s * SUB, SUB)], sem)


def _sc_kernel(lp_ref, bias_ref, os_ref, oi_ref, ob_ref,
               dbuf, biasv, valbuf, idxbuf, bmaxbuf,
               ov, oi, ob, sem0, sem1):
    wid = lax.axis_index("c") * NS + lax.axis_index("s")
    pltpu.sync_copy(bias_ref, biasv)
    row0 = wid * 8

    _issue_chunk(lp_ref, dbuf, sem0, row0, 0, 0)

    neg_inf = jnp.full((L,), -jnp.inf, jnp.float32)
    zeros_i = jnp.zeros((L,), jnp.int32)

    def row_body(r, dummy):
        row = row0 + r
        bias = plsc.load_gather(biasv, [jnp.broadcast_to(row, (L,))
                                        .astype(jnp.int32)])
        _issue_chunk(lp_ref, dbuf, sem1, row, 1, 1)
        pltpu.make_async_copy(lp_ref.at[row, pl.ds(0, CHUNK)],
                              dbuf.at[pl.ds(0, CHUNK)], sem0).wait()
        carry = _scan_chunk(dbuf, bmaxbuf, 0, bias,
                            (neg_inf, zeros_i, neg_inf))
        nrow = row0 + ((r + 1) & 7)
        _issue_chunk(lp_ref, dbuf, sem0, nrow, 0, 0)
        pltpu.make_async_copy(lp_ref.at[row, pl.ds(CHUNK, CHUNK)],
                              dbuf.at[pl.ds(CHUNK, CHUNK)], sem1).wait()
        rv, ri, _ = _scan_chunk(dbuf, bmaxbuf, 1, bias, carry)
        valbuf[pl.ds(r * K, K)] = rv
        idxbuf[pl.ds(r * K, K)] = ri
        return dummy

    lax.fori_loop(0, 8, row_body, jnp.int32(0))
    pltpu.make_async_copy(lp_ref.at[row0, pl.ds(0, CHUNK)],
                          dbuf.at[pl.ds(0, CHUNK)], sem0).wait()

    pen = (_iota16().astype(jnp.float32) + 1.0) * DIVERSITY_RATE
    r2v, r2i = neg_inf, zeros_i
    for b in range(8):
        pv = valbuf[pl.ds(b * K, K)] - pen
        fi = jnp.broadcast_to(jnp.int32(b * K), (L,)) + _iota16()
        r2v, r2i, _ = _merge_top16(r2v, r2i, pv, fi)

    ov[...] = r2v
    oi[...] = plsc.load_gather(idxbuf, [r2i])
    ob[...] = jnp.right_shift(r2i, 4)
    pltpu.sync_copy(ov, os_ref.at[wid])
    pltpu.sync_copy(oi, oi_ref.at[wid])
    pltpu.sync_copy(ob, ob_ref.at[wid])


def kernel(lprobs, scores, step):
    bsz, beam, vocab = lprobs.shape
    assert (bsz, beam, vocab) == (32, 8, V)
    lp2 = lprobs.reshape(bsz * beam, vocab)
    bias = lax.dynamic_index_in_dim(scores, step - 1, axis=2,
                                    keepdims=False).reshape(bsz * beam)

    mesh = plsc.VectorSubcoreMesh(core_axis_name="c", subcore_axis_name="s",
                                  num_cores=NC, num_subcores=NS)
    f = pl.kernel(
        _sc_kernel,
        out_type=(
            jax.ShapeDtypeStruct((bsz, K), jnp.float32),
            jax.ShapeDtypeStruct((bsz, K), jnp.int32),
            jax.ShapeDtypeStruct((bsz, K), jnp.int32),
        ),
        mesh=mesh,
        compiler_params=pltpu.CompilerParams(use_tc_tiling_on_sc=False,
                                             needs_layout_passes=False),
        scratch_types=[
            pltpu.VMEM((V,), jnp.float32),
            pltpu.VMEM((bsz * beam,), jnp.float32),
            pltpu.VMEM((beam * K,), jnp.float32),
            pltpu.VMEM((beam * K,), jnp.int32),
            pltpu.VMEM((128 * L,), jnp.float32),
            pltpu.VMEM((K,), jnp.float32),
            pltpu.VMEM((K,), jnp.int32),
            pltpu.VMEM((K,), jnp.int32),
            pltpu.SemaphoreType.DMA,
            pltpu.SemaphoreType.DMA,
        ],
    )
    return f(lp2, bias)
